# baseline (speedup 1.0000x reference)
import functools

import jax
import jax.numpy as jnp
from jax import lax
from jax.experimental import pallas as pl
from jax.experimental.pallas import tpu as pltpu
from jax.experimental.pallas import tpu_sc as plsc

LANES = 128
BLK = 1280
NSC = 2
NTILES = 16


def _sc_mesh():
    return plsc.VectorSubcoreMesh(core_axis_name="c", subcore_axis_name="s")


def _sc_degrees(dst2d, src2d, zeros1d, n_pad):
    er = dst2d.shape[0]
    rows_per_tile = er // NTILES
    stripe = n_pad // NTILES

    @functools.partial(
        pl.kernel,
        out_type=jax.ShapeDtypeStruct((NSC, n_pad), jnp.float32),
        mesh=_sc_mesh(),
        scratch_types=[
            pltpu.VMEM((rows_per_tile, LANES), jnp.int32),
            pltpu.VMEM((LANES,), jnp.float32),
            pltpu.VMEM_SHARED((n_pad,), jnp.float32),
        ],
    )
    def k(dst_hbm, src_hbm, zeros_hbm, out_hbm, idx_v, ones_v, accum):
        c = lax.axis_index("c")
        s = lax.axis_index("s")
        for j in range(LANES // 16):
            ones_v[pl.ds(j * 16, 16)] = jnp.ones((16,), jnp.float32)
        pltpu.sync_copy(zeros_hbm.at[pl.ds(s * stripe, stripe)],
                        accum.at[pl.ds(s * stripe, stripe)])

        @pl.when(c == 0)
        def _():
            pltpu.sync_copy(dst_hbm.at[pl.ds(s * rows_per_tile, rows_per_tile)],
                            idx_v)

        @pl.when(c == 1)
        def _():
            pltpu.sync_copy(src_hbm.at[pl.ds(s * rows_per_tile, rows_per_tile)],
                            idx_v)

        plsc.subcore_barrier()

        def body(j, carry):
            pltpu.sync_copy(ones_v, accum.at[idx_v.at[j]], add=True)
            return carry

        lax.fori_loop(0, rows_per_tile, body, 0)
        plsc.subcore_barrier()
        pltpu.sync_copy(accum.at[pl.ds(s * stripe, stripe)],
                        out_hbm.at[c, pl.ds(s * stripe, stripe)])

    return k(dst2d, src2d, zeros1d)


SEG = 16


def _spmm_phase(tab, src_v, loc_v, rows2, gsem0, gsem1, accum, nchunks):
    buf0 = rows2.at[0]
    buf1 = rows2.at[1]

    def gstart(i, buf, sem):
        pltpu.async_copy(tab.at[src_v.at[i]], buf, sem)

    def gwait(i, buf, sem):
        pltpu.make_async_copy(tab.at[src_v.at[i]], buf, sem).wait()

    def scat(i, buf):
        pltpu.sync_copy(buf, accum.at[loc_v.at[i]], add=True)

    gstart(0, buf0, gsem0)

    def body(k, carry):
        i0 = 2 * k
        gstart(i0 + 1, buf1, gsem1)
        gwait(i0, buf0, gsem0)
        scat(i0, buf0)
        gstart(i0 + 2, buf0, gsem0)
        gwait(i0 + 1, buf1, gsem1)
        scat(i0 + 1, buf1)
        return carry

    lax.fori_loop(0, nchunks // 2 - 1, body, 0)
    i0 = nchunks - 2
    gstart(i0 + 1, buf1, gsem1)
    gwait(i0, buf0, gsem0)
    scat(i0, buf0)
    gwait(i0 + 1, buf1, gsem1)
    scat(i0 + 1, buf1)


def _zero_accum(zeros_hbm, accum, n_pad, s):
    zstripe = n_pad // NTILES
    pltpu.sync_copy(zeros_hbm.at[pl.ds(s * zstripe, zstripe)],
                    accum.at[pl.ds(s * zstripe, zstripe)])


def _flush_accum(accum, out_hbm, t, n_pad, s):
    fstripe = n_pad // NTILES
    pltpu.sync_copy(
        accum.at[pl.ds(s * fstripe, fstripe)],
        out_hbm.at[t, pl.ds(s * fstripe, fstripe)])


def _spmm_streamed(tab, src_hbm, dst_hbm, idx4, isems, rows2, gsem0, gsem1,
                   accum, base, rows):
    nseg = rows // SEG

    def istart(g, p):
        off = base + g * SEG
        pltpu.async_copy(src_hbm.at[pl.ds(off, SEG)], idx4.at[p, 0],
                         isems[2 * p])
        pltpu.async_copy(dst_hbm.at[pl.ds(off, SEG)], idx4.at[p, 1],
                         isems[2 * p + 1])

    def iwait(g, p):
        off = base + g * SEG
        pltpu.make_async_copy(src_hbm.at[pl.ds(off, SEG)], idx4.at[p, 0],
                              isems[2 * p]).wait()
        pltpu.make_async_copy(dst_hbm.at[pl.ds(off, SEG)], idx4.at[p, 1],
                              isems[2 * p + 1]).wait()

    istart(0, 0)
    for g in range(nseg):
        p = g % 2
        iwait(g, p)
        if g + 1 < nseg:
            istart(g + 1, 1 - p)
        _spmm_phase(tab, idx4.at[p, 0], idx4.at[p, 1], rows2, gsem0, gsem1,
                    accum, SEG)


def _sc_spmm_split(xprime, src2d, dst2d, zeros2d, n_pad):
    er = src2d.shape[0]
    rows_per_tile = er // (NSC * NTILES)

    @functools.partial(
        pl.kernel,
        out_type=jax.ShapeDtypeStruct((NSC, n_pad, LANES), jnp.float32),
        mesh=_sc_mesh(),
        scratch_types=[
            pltpu.VMEM((2, 2, SEG, LANES), jnp.int32),
            pltpu.VMEM((2, LANES, LANES), jnp.float32),
            pltpu.SemaphoreType.DMA,
            pltpu.SemaphoreType.DMA,
            pltpu.SemaphoreType.DMA,
            pltpu.SemaphoreType.DMA,
            pltpu.SemaphoreType.DMA,
            pltpu.SemaphoreType.DMA,
            pltpu.VMEM_SHARED((n_pad, LANES), jnp.float32),
        ],
    )
    def k(x_hbm, src_hbm, dst_hbm, zeros_hbm, out_hbm, idx4,
          rows2, is0, is1, is2, is3, gsem0, gsem1, accum):
        c = lax.axis_index("c")
        s = lax.axis_index("s")
        wid = c * NTILES + s
        _zero_accum(zeros_hbm, accum, n_pad, s)
        plsc.subcore_barrier()
        _spmm_streamed(x_hbm.at[c], src_hbm, dst_hbm, idx4,
                       (is0, is1, is2, is3), rows2,
                       gsem0, gsem1, accum, wid * rows_per_tile,
                       rows_per_tile)
        plsc.subcore_barrier()
        _flush_accum(accum, out_hbm, c, n_pad, s)

    return k(xprime, src2d, dst2d, zeros2d)


def _sc_spmm_dual(y0, logits, src2d, dst2d, zeros2d, n_pad):
    er = src2d.shape[0]
    rows_per_tile = er // NTILES

    @functools.partial(
        pl.kernel,
        out_type=jax.ShapeDtypeStruct((NSC, n_pad, LANES), jnp.float32),
        mesh=_sc_mesh(),
        scratch_types=[
            pltpu.VMEM((2, 2, SEG, LANES), jnp.int32),
            pltpu.VMEM((2, LANES, LANES), jnp.float32),
            pltpu.SemaphoreType.DMA,
            pltpu.SemaphoreType.DMA,
            pltpu.SemaphoreType.DMA,
            pltpu.SemaphoreType.DMA,
            pltpu.SemaphoreType.DMA,
            pltpu.SemaphoreType.DMA,
            pltpu.VMEM_SHARED((n_pad, LANES), jnp.float32),
        ],
    )
    def k(y0_hbm, lg_hbm, src_hbm, dst_hbm, zeros_hbm, out_hbm, idx4,
          rows2, is0, is1, is2, is3, gsem0, gsem1, accum):
        c = lax.axis_index("c")
        s = lax.axis_index("s")
        _zero_accum(zeros_hbm, accum, n_pad, s)
        plsc.subcore_barrier()

        @pl.when(c == 0)
        def _():
            _spmm_streamed(y0_hbm, src_hbm, dst_hbm, idx4,
                           (is0, is1, is2, is3), rows2,
                           gsem0, gsem1, accum, s * rows_per_tile,
                           rows_per_tile)

        @pl.when(c == 1)
        def _():
            _spmm_streamed(lg_hbm, src_hbm, dst_hbm, idx4,
                           (is0, is1, is2, is3), rows2,
                           gsem0, gsem1, accum, s * rows_per_tile,
                           rows_per_tile)

        plsc.subcore_barrier()
        _flush_accum(accum, out_hbm, c, n_pad, s)

    return k(y0, logits, src2d, dst2d, zeros2d)


def _tc_scale_x(x_pad, deg_out_col, n_pad):
    grid = n_pad // BLK

    def body(do_ref, x_ref, xp_ref):
        d = jnp.maximum(do_ref[...], 1.0)
        xp = x_ref[...] * lax.rsqrt(d)
        xp_ref[0] = xp
        xp_ref[1] = xp

    return pl.pallas_call(
        body,
        grid=(grid,),
        in_specs=[
            pl.BlockSpec((BLK, 1), lambda i: (i, 0)),
            pl.BlockSpec((BLK, LANES), lambda i: (i, 0)),
        ],
        out_specs=pl.BlockSpec((NSC, BLK, LANES), lambda i: (0, i, 0)),
        out_shape=jax.ShapeDtypeStruct((NSC, n_pad, LANES), jnp.float32),
    )(deg_out_col, x_pad)


def _tc_mid(s1, deg_in_col, deg_out_col, W0, lin_W, n_pad):
    grid = n_pad // BLK

    def body(s1_ref, di_ref, do_ref, w0_ref, lw_ref, h0_ref, y0_ref, lg_ref):
        ssum = s1_ref[0] + s1_ref[1]
        din = jnp.maximum(di_ref[...], 1.0)
        dout = jnp.maximum(do_ref[...], 1.0)
        h0 = jnp.dot(ssum * lax.rsqrt(din), w0_ref[...],
                     preferred_element_type=jnp.float32)
        t = jnp.dot(h0, lw_ref[...], preferred_element_type=jnp.float32)
        t = t - jnp.max(t, axis=1, keepdims=True)
        e = jnp.exp(t)
        lg = e / jnp.sum(e, axis=1, keepdims=True)
        h0_ref[...] = h0
        y0_ref[...] = h0 * lax.rsqrt(dout)
        lg_ref[...] = lg

    return pl.pallas_call(
        body,
        grid=(grid,),
        in_specs=[
            pl.BlockSpec((NSC, BLK, LANES), lambda i: (0, i, 0)),
            pl.BlockSpec((BLK, 1), lambda i: (i, 0)),
            pl.BlockSpec((BLK, 1), lambda i: (i, 0)),
            pl.BlockSpec((LANES, LANES), lambda i: (0, 0)),
            pl.BlockSpec((LANES, LANES), lambda i: (0, 0)),
        ],
        out_specs=[
            pl.BlockSpec((BLK, LANES), lambda i: (i, 0)),
            pl.BlockSpec((BLK, LANES), lambda i: (i, 0)),
            pl.BlockSpec((BLK, LANES), lambda i: (i, 0)),
        ],
        out_shape=[
            jax.ShapeDtypeStruct((n_pad, LANES), jnp.float32),
            jax.ShapeDtypeStruct((n_pad, LANES), jnp.float32),
            jax.ShapeDtypeStruct((n_pad, LANES), jnp.float32),
        ],
    )(s1, deg_in_col, deg_out_col, W0, lin_W)


def _tc_final(s2, h0, logits, deg_in_col, W1, lin_W, tau1, tau2, n_pad):
    grid = n_pad // BLK

    def body(s2_ref, h0_ref, lg_ref, di_ref, w1_ref, lw_ref, t1_ref, t2_ref,
             out_ref):
        din = jnp.maximum(di_ref[...], 1.0)
        agg = s2_ref[0] * lax.rsqrt(din)
        neigh = s2_ref[1] / din
        lg = lg_ref[...]
        score = jnp.sum(lg * neigh, axis=1, keepdims=True)
        z = jax.nn.sigmoid(t1_ref[0, 0] * score + t2_ref[0, 0])
        h1 = z * agg + (1.0 - z) * h0_ref[...]
        w = jnp.dot(w1_ref[...], lw_ref[...],
                    preferred_element_type=jnp.float32)
        out_ref[...] = jnp.dot(h1, w, preferred_element_type=jnp.float32)

    return pl.pallas_call(
        body,
        grid=(grid,),
        in_specs=[
            pl.BlockSpec((NSC, BLK, LANES), lambda i: (0, i, 0)),
            pl.BlockSpec((BLK, LANES), lambda i: (i, 0)),
            pl.BlockSpec((BLK, LANES), lambda i: (i, 0)),
            pl.BlockSpec((BLK, 1), lambda i: (i, 0)),
            pl.BlockSpec((LANES, LANES), lambda i: (0, 0)),
            pl.BlockSpec((LANES, LANES), lambda i: (0, 0)),
            pl.BlockSpec(memory_space=pltpu.SMEM),
            pl.BlockSpec(memory_space=pltpu.SMEM),
        ],
        out_specs=pl.BlockSpec((BLK, LANES), lambda i: (i, 0)),
        out_shape=jax.ShapeDtypeStruct((n_pad, LANES), jnp.float32),
    )(s2, h0, logits, deg_in_col, W1, lin_W, tau1, tau2)


def kernel(x, edge_index, W0, W1, lin_W, init_weight_y, tau1, tau2):
    n, d = x.shape
    e = edge_index.shape[1]
    assert d == LANES

    n_pad = ((n + 1 + BLK - 1) // BLK) * BLK

    er = (e + LANES - 1) // LANES
    er_pad = ((er + 8 * NSC * NTILES - 1) // (8 * NSC * NTILES)) * (8 * NSC * NTILES)
    e_pad = er_pad * LANES

    src = edge_index[0]
    dst = edge_index[1]
    pad = n + (jnp.arange(e_pad - e, dtype=jnp.int32) % (n_pad - n))
    src2d = jnp.concatenate([src, pad]).reshape(er_pad, LANES)
    dst2d = jnp.concatenate([dst, pad]).reshape(er_pad, LANES)

    x_pad = jnp.zeros((n_pad, LANES), jnp.float32).at[:n].set(x)
    zeros2d = jnp.zeros((n_pad, LANES), jnp.float32)
    zeros1d = jnp.zeros((n_pad,), jnp.float32)

    degs = _sc_degrees(dst2d, src2d, zeros1d, n_pad)
    deg_in_col = degs[0].reshape(n_pad, 1)
    deg_out_col = degs[1].reshape(n_pad, 1)

    xprime = _tc_scale_x(x_pad, deg_out_col, n_pad)
    s1 = _sc_spmm_split(xprime, src2d, dst2d, zeros2d, n_pad)
    h0, y0, logits = _tc_mid(s1, deg_in_col, deg_out_col, W0, lin_W, n_pad)
    s2 = _sc_spmm_dual(y0, logits, src2d, dst2d, zeros2d, n_pad)
    out = _tc_final(s2, h0, logits, deg_in_col, W1, lin_W,
                    tau1.reshape(1, 1), tau2.reshape(1, 1), n_pad)
    return out[:n]


# speedup vs baseline: 1.0323x; 1.0323x over previous
import functools

import jax
import jax.numpy as jnp
from jax import lax
from jax.experimental import pallas as pl
from jax.experimental.pallas import tpu as pltpu
from jax.experimental.pallas import tpu_sc as plsc

LANES = 128
BLK = 1280
NSC = 2
NTILES = 16


def _sc_mesh():
    return plsc.VectorSubcoreMesh(core_axis_name="c", subcore_axis_name="s")


def _sc_degrees(dst2d, src2d, zeros1d, n_pad):
    er = dst2d.shape[0]
    rows_per_tile = er // NTILES
    stripe = n_pad // NTILES

    @functools.partial(
        pl.kernel,
        out_type=jax.ShapeDtypeStruct((NSC, n_pad), jnp.float32),
        mesh=_sc_mesh(),
        scratch_types=[
            pltpu.VMEM((rows_per_tile, LANES), jnp.int32),
            pltpu.VMEM((LANES,), jnp.float32),
            pltpu.VMEM_SHARED((n_pad,), jnp.float32),
        ],
    )
    def k(dst_hbm, src_hbm, zeros_hbm, out_hbm, idx_v, ones_v, accum):
        c = lax.axis_index("c")
        s = lax.axis_index("s")
        for j in range(LANES // 16):
            ones_v[pl.ds(j * 16, 16)] = jnp.ones((16,), jnp.float32)
        pltpu.sync_copy(zeros_hbm.at[pl.ds(s * stripe, stripe)],
                        accum.at[pl.ds(s * stripe, stripe)])

        @pl.when(c == 0)
        def _():
            pltpu.sync_copy(dst_hbm.at[pl.ds(s * rows_per_tile, rows_per_tile)],
                            idx_v)

        @pl.when(c == 1)
        def _():
            pltpu.sync_copy(src_hbm.at[pl.ds(s * rows_per_tile, rows_per_tile)],
                            idx_v)

        plsc.subcore_barrier()

        def body(j, carry):
            pltpu.sync_copy(ones_v, accum.at[idx_v.at[j]], add=True)
            return carry

        lax.fori_loop(0, rows_per_tile, body, 0)
        plsc.subcore_barrier()
        pltpu.sync_copy(accum.at[pl.ds(s * stripe, stripe)],
                        out_hbm.at[c, pl.ds(s * stripe, stripe)])

    return k(dst2d, src2d, zeros1d)


SEG = 40


def _spmm_phase(tab, src_v, loc_v, rows2, gsem0, gsem1, accum, nchunks):
    buf0 = rows2.at[0]
    buf1 = rows2.at[1]

    def gstart(i, buf, sem):
        pltpu.async_copy(tab.at[src_v.at[i]], buf, sem)

    def gwait(i, buf, sem):
        pltpu.make_async_copy(tab.at[src_v.at[i]], buf, sem).wait()

    def scat(i, buf):
        pltpu.sync_copy(buf, accum.at[loc_v.at[i]], add=True)

    gstart(0, buf0, gsem0)

    def body(k, carry):
        i0 = 2 * k
        gstart(i0 + 1, buf1, gsem1)
        gwait(i0, buf0, gsem0)
        scat(i0, buf0)
        gstart(i0 + 2, buf0, gsem0)
        gwait(i0 + 1, buf1, gsem1)
        scat(i0 + 1, buf1)
        return carry

    lax.fori_loop(0, nchunks // 2 - 1, body, 0)
    i0 = nchunks - 2
    gstart(i0 + 1, buf1, gsem1)
    gwait(i0, buf0, gsem0)
    scat(i0, buf0)
    gwait(i0 + 1, buf1, gsem1)
    scat(i0 + 1, buf1)


def _zero_accum(zeros_hbm, accum, n_pad, s):
    zstripe = n_pad // NTILES
    pltpu.sync_copy(zeros_hbm.at[pl.ds(s * zstripe, zstripe)],
                    accum.at[pl.ds(s * zstripe, zstripe)])


def _flush_accum(accum, out_hbm, t, n_pad, s):
    fstripe = n_pad // NTILES
    pltpu.sync_copy(
        accum.at[pl.ds(s * fstripe, fstripe)],
        out_hbm.at[t, pl.ds(s * fstripe, fstripe)])


def _spmm_streamed(tab, src_hbm, dst_hbm, src_v, dst_v, rows2, gsem0, gsem1,
                   accum, base, rows):
    def seg_body(g, carry):
        off = base + g * SEG
        pltpu.sync_copy(src_hbm.at[pl.ds(off, SEG)], src_v)
        pltpu.sync_copy(dst_hbm.at[pl.ds(off, SEG)], dst_v)
        _spmm_phase(tab, src_v, dst_v, rows2, gsem0, gsem1, accum, SEG)
        return carry

    lax.fori_loop(0, rows // SEG, seg_body, 0)


def _sc_spmm_split(xprime, src2d, dst2d, zeros2d, n_pad):
    er = src2d.shape[0]
    rows_per_tile = er // (NSC * NTILES)

    @functools.partial(
        pl.kernel,
        out_type=jax.ShapeDtypeStruct((NSC, n_pad, LANES), jnp.float32),
        mesh=_sc_mesh(),
        scratch_types=[
            pltpu.VMEM((SEG, LANES), jnp.int32),
            pltpu.VMEM((SEG, LANES), jnp.int32),
            pltpu.VMEM((2, LANES, LANES), jnp.float32),
            pltpu.SemaphoreType.DMA,
            pltpu.SemaphoreType.DMA,
            pltpu.VMEM_SHARED((n_pad, LANES), jnp.float32),
        ],
    )
    def k(x_hbm, src_hbm, dst_hbm, zeros_hbm, out_hbm, src_v, dst_v,
          rows2, gsem0, gsem1, accum):
        c = lax.axis_index("c")
        s = lax.axis_index("s")
        wid = c * NTILES + s
        _zero_accum(zeros_hbm, accum, n_pad, s)
        plsc.subcore_barrier()
        _spmm_streamed(x_hbm.at[c], src_hbm, dst_hbm, src_v, dst_v, rows2,
                       gsem0, gsem1, accum, wid * rows_per_tile,
                       rows_per_tile)
        plsc.subcore_barrier()
        _flush_accum(accum, out_hbm, c, n_pad, s)

    return k(xprime, src2d, dst2d, zeros2d)


def _sc_spmm_dual(y0, logits, src2d, dst2d, zeros2d, n_pad):
    er = src2d.shape[0]
    rows_per_tile = er // NTILES

    @functools.partial(
        pl.kernel,
        out_type=jax.ShapeDtypeStruct((NSC, n_pad, LANES), jnp.float32),
        mesh=_sc_mesh(),
        scratch_types=[
            pltpu.VMEM((SEG, LANES), jnp.int32),
            pltpu.VMEM((SEG, LANES), jnp.int32),
            pltpu.VMEM((2, LANES, LANES), jnp.float32),
            pltpu.SemaphoreType.DMA,
            pltpu.SemaphoreType.DMA,
            pltpu.VMEM_SHARED((n_pad, LANES), jnp.float32),
        ],
    )
    def k(y0_hbm, lg_hbm, src_hbm, dst_hbm, zeros_hbm, out_hbm, src_v, dst_v,
          rows2, gsem0, gsem1, accum):
        c = lax.axis_index("c")
        s = lax.axis_index("s")
        _zero_accum(zeros_hbm, accum, n_pad, s)
        plsc.subcore_barrier()

        @pl.when(c == 0)
        def _():
            _spmm_streamed(y0_hbm, src_hbm, dst_hbm, src_v, dst_v, rows2,
                           gsem0, gsem1, accum, s * rows_per_tile,
                           rows_per_tile)

        @pl.when(c == 1)
        def _():
            _spmm_streamed(lg_hbm, src_hbm, dst_hbm, src_v, dst_v, rows2,
                           gsem0, gsem1, accum, s * rows_per_tile,
                           rows_per_tile)

        plsc.subcore_barrier()
        _flush_accum(accum, out_hbm, c, n_pad, s)

    return k(y0, logits, src2d, dst2d, zeros2d)


def _tc_scale_x(x_pad, deg_out_col, n_pad):
    grid = n_pad // BLK

    def body(do_ref, x_ref, xp_ref):
        d = jnp.maximum(do_ref[...], 1.0)
        xp = x_ref[...] * lax.rsqrt(d)
        xp_ref[0] = xp
        xp_ref[1] = xp

    return pl.pallas_call(
        body,
        grid=(grid,),
        in_specs=[
            pl.BlockSpec((BLK, 1), lambda i: (i, 0)),
            pl.BlockSpec((BLK, LANES), lambda i: (i, 0)),
        ],
        out_specs=pl.BlockSpec((NSC, BLK, LANES), lambda i: (0, i, 0)),
        out_shape=jax.ShapeDtypeStruct((NSC, n_pad, LANES), jnp.float32),
    )(deg_out_col, x_pad)


def _tc_mid(s1, deg_in_col, deg_out_col, W0, lin_W, n_pad):
    grid = n_pad // BLK

    def body(s1_ref, di_ref, do_ref, w0_ref, lw_ref, h0_ref, y0_ref, lg_ref):
        ssum = s1_ref[0] + s1_ref[1]
        din = jnp.maximum(di_ref[...], 1.0)
        dout = jnp.maximum(do_ref[...], 1.0)
        h0 = jnp.dot(ssum * lax.rsqrt(din), w0_ref[...],
                     preferred_element_type=jnp.float32)
        t = jnp.dot(h0, lw_ref[...], preferred_element_type=jnp.float32)
        t = t - jnp.max(t, axis=1, keepdims=True)
        e = jnp.exp(t)
        lg = e / jnp.sum(e, axis=1, keepdims=True)
        h0_ref[...] = h0
        y0_ref[...] = h0 * lax.rsqrt(dout)
        lg_ref[...] = lg

    return pl.pallas_call(
        body,
        grid=(grid,),
        in_specs=[
            pl.BlockSpec((NSC, BLK, LANES), lambda i: (0, i, 0)),
            pl.BlockSpec((BLK, 1), lambda i: (i, 0)),
            pl.BlockSpec((BLK, 1), lambda i: (i, 0)),
            pl.BlockSpec((LANES, LANES), lambda i: (0, 0)),
            pl.BlockSpec((LANES, LANES), lambda i: (0, 0)),
        ],
        out_specs=[
            pl.BlockSpec((BLK, LANES), lambda i: (i, 0)),
            pl.BlockSpec((BLK, LANES), lambda i: (i, 0)),
            pl.BlockSpec((BLK, LANES), lambda i: (i, 0)),
        ],
        out_shape=[
            jax.ShapeDtypeStruct((n_pad, LANES), jnp.float32),
            jax.ShapeDtypeStruct((n_pad, LANES), jnp.float32),
            jax.ShapeDtypeStruct((n_pad, LANES), jnp.float32),
        ],
    )(s1, deg_in_col, deg_out_col, W0, lin_W)


def _tc_final(s2, h0, logits, deg_in_col, W1, lin_W, tau1, tau2, n_pad):
    grid = n_pad // BLK

    def body(s2_ref, h0_ref, lg_ref, di_ref, w1_ref, lw_ref, t1_ref, t2_ref,
             out_ref):
        din = jnp.maximum(di_ref[...], 1.0)
        agg = s2_ref[0] * lax.rsqrt(din)
        neigh = s2_ref[1] / din
        lg = lg_ref[...]
        score = jnp.sum(lg * neigh, axis=1, keepdims=True)
        z = jax.nn.sigmoid(t1_ref[0, 0] * score + t2_ref[0, 0])
        h1 = z * agg + (1.0 - z) * h0_ref[...]
        w = jnp.dot(w1_ref[...], lw_ref[...],
                    preferred_element_type=jnp.float32)
        out_ref[...] = jnp.dot(h1, w, preferred_element_type=jnp.float32)

    return pl.pallas_call(
        body,
        grid=(grid,),
        in_specs=[
            pl.BlockSpec((NSC, BLK, LANES), lambda i: (0, i, 0)),
            pl.BlockSpec((BLK, LANES), lambda i: (i, 0)),
            pl.BlockSpec((BLK, LANES), lambda i: (i, 0)),
            pl.BlockSpec((BLK, 1), lambda i: (i, 0)),
            pl.BlockSpec((LANES, LANES), lambda i: (0, 0)),
            pl.BlockSpec((LANES, LANES), lambda i: (0, 0)),
            pl.BlockSpec(memory_space=pltpu.SMEM),
            pl.BlockSpec(memory_space=pltpu.SMEM),
        ],
        out_specs=pl.BlockSpec((BLK, LANES), lambda i: (i, 0)),
        out_shape=jax.ShapeDtypeStruct((n_pad, LANES), jnp.float32),
    )(s2, h0, logits, deg_in_col, W1, lin_W, tau1, tau2)


def kernel(x, edge_index, W0, W1, lin_W, init_weight_y, tau1, tau2):
    n, d = x.shape
    e = edge_index.shape[1]
    assert d == LANES

    n_pad = ((n + 1 + BLK - 1) // BLK) * BLK

    er = (e + LANES - 1) // LANES
    er_pad = ((er + 8 * NSC * NTILES - 1) // (8 * NSC * NTILES)) * (8 * NSC * NTILES)
    e_pad = er_pad * LANES

    src = edge_index[0]
    dst = edge_index[1]
    pad = n + (jnp.arange(e_pad - e, dtype=jnp.int32) % (n_pad - n))
    src2d = jnp.concatenate([src, pad]).reshape(er_pad, LANES)
    dst2d = jnp.concatenate([dst, pad]).reshape(er_pad, LANES)

    x_pad = jnp.zeros((n_pad, LANES), jnp.float32).at[:n].set(x)
    zeros2d = jnp.zeros((n_pad, LANES), jnp.float32)
    zeros1d = jnp.zeros((n_pad,), jnp.float32)

    degs = _sc_degrees(dst2d, src2d, zeros1d, n_pad)
    deg_in_col = degs[0].reshape(n_pad, 1)
    deg_out_col = degs[1].reshape(n_pad, 1)

    xprime = _tc_scale_x(x_pad, deg_out_col, n_pad)
    s1 = _sc_spmm_split(xprime, src2d, dst2d, zeros2d, n_pad)
    h0, y0, logits = _tc_mid(s1, deg_in_col, deg_out_col, W0, lin_W, n_pad)
    s2 = _sc_spmm_dual(y0, logits, src2d, dst2d, zeros2d, n_pad)
    out = _tc_final(s2, h0, logits, deg_in_col, W1, lin_W,
                    tau1.reshape(1, 1), tau2.reshape(1, 1), n_pad)
    return out[:n]
